# baseline (speedup 1.0000x reference)
import functools

import jax
import jax.numpy as jnp
from jax import lax
from jax.experimental import pallas as pl
from jax.experimental.pallas import tpu as pltpu
from jax.experimental.pallas import tpu_sc as plsc

BATCH = 1024
NT = 200
D = 128
ROWS = BATCH * NT
NC, NS, L = 2, 16, 16
NW = NC * NS
B_PER_W = ROWS // NW
CHUNK = 128
NCHUNK = B_PER_W // CHUNK
VECS = CHUNK // L
PB = 2
NPB = NCHUNK // PB
NBUF = 3
TBL_ROWS = 158


def _sc_gather(idx_hbm, seq_hbm, table_hbm):
    mesh = plsc.VectorSubcoreMesh(core_axis_name="c", subcore_axis_name="s")

    @functools.partial(
        pl.kernel,
        out_type=jax.ShapeDtypeStruct((ROWS, D), jnp.float32),
        mesh=mesh,
        scratch_types=[
            pltpu.VMEM((NCHUNK, CHUNK), jnp.int32),
            pltpu.VMEM((NBUF, PB * CHUNK, D), jnp.float32),
            pltpu.VMEM((L,), jnp.int32),
            pltpu.VMEM_SHARED((TBL_ROWS, D), jnp.float32),
            pltpu.SemaphoreType.DMA((NBUF,)),
            pltpu.SemaphoreType.DMA((NBUF,)),
        ],
    )
    def body(idx_ref, seq_ref, tbl_ref, out_ref,
             idx_v, rows_v, seq_v, tbl_sh, gsem, psem):
        sid = lax.axis_index("s")
        wid = sid * NC + lax.axis_index("c")
        base = wid * B_PER_W

        @pl.when(sid == 0)
        def _():
            pltpu.sync_copy(tbl_ref, tbl_sh)

        pltpu.sync_copy(idx_ref.at[wid], idx_v)
        pltpu.sync_copy(seq_ref, seq_v)
        seq = seq_v[...]
        lane = lax.iota(jnp.int32, L)

        def xform(j, _):
            for k in range(VECS):
                pos0 = base + j * CHUNK + k * L
                t = lax.rem(pos0 + lane, NT)
                v = idx_v[j, pl.ds(k * L, L)]
                idx_v[j, pl.ds(k * L, L)] = jnp.where(t < seq, v + 1, 0)
            return 0

        lax.fori_loop(0, NCHUNK, xform, 0)
        plsc.subcore_barrier()

        def gather_cp(b, p, j):
            return pltpu.make_async_copy(
                tbl_sh.at[idx_v.at[j]],
                rows_v.at[b, pl.ds(p * CHUNK, CHUNK)], gsem.at[b])

        def start_gathers(b, jb):
            for p in range(PB):
                gather_cp(b, p, jb * PB + p).start()

        def wait_gathers(b, jb):
            for p in range(PB):
                gather_cp(b, p, jb * PB + p).wait()

        def put_cp(b, jb):
            return pltpu.make_async_copy(
                rows_v.at[b],
                out_ref.at[pl.ds(base + jb * PB * CHUNK, PB * CHUNK)],
                psem.at[b])

        for b in range(NBUF):
            start_gathers(b, b)

        nfull = (NPB // NBUF) * NBUF

        def round_body(r, _):
            for b in range(NBUF):
                jb = r * NBUF + b
                wait_gathers(b, jb)
                put_cp(b, jb).start()
                jn = jb + NBUF

                @pl.when(jn < NPB)
                def _():
                    put_cp(b, jb).wait()
                    start_gathers(b, jn)
            return 0

        lax.fori_loop(0, nfull // NBUF, round_body, 0)

        for jb in range(nfull, NPB):
            b = jb % NBUF
            wait_gathers(b, jb)
            put_cp(b, jb).start()
        for jb in range(NPB - NBUF, NPB):
            put_cp(jb % NBUF, jb).wait()

    return body(idx_hbm, seq_hbm, table_hbm)


def kernel(text, seq_len, text_embed, text_embed_ko):
    del text_embed
    idx = text.reshape(NW, NCHUNK, CHUNK).astype(jnp.int32)
    seq = jnp.full((L,), seq_len, dtype=jnp.int32)
    out = _sc_gather(idx, seq, text_embed_ko)
    return out.reshape(BATCH, NT, D)


# speedup vs baseline: 1.0078x; 1.0078x over previous
import functools

import jax
import jax.numpy as jnp
from jax import lax
from jax.experimental import pallas as pl
from jax.experimental.pallas import tpu as pltpu
from jax.experimental.pallas import tpu_sc as plsc

BATCH = 1024
NT = 200
D = 128
ROWS = BATCH * NT
NC, NS, L = 2, 16, 16
NW = NC * NS
B_PER_W = ROWS // NW
CHUNK = 128
NCHUNK = B_PER_W // CHUNK
VECS = CHUNK // L
NBUF = 5
ROUNDS = NCHUNK // NBUF
TBL_ROWS = 158


def _sc_gather(idx_hbm, seq_hbm, table_hbm):
    mesh = plsc.VectorSubcoreMesh(core_axis_name="c", subcore_axis_name="s")

    @functools.partial(
        pl.kernel,
        out_type=jax.ShapeDtypeStruct((ROWS, D), jnp.float32),
        mesh=mesh,
        scratch_types=[
            pltpu.VMEM((NCHUNK, CHUNK), jnp.int32),
            pltpu.VMEM((NBUF, CHUNK, D), jnp.float32),
            pltpu.VMEM((L,), jnp.int32),
            pltpu.VMEM_SHARED((TBL_ROWS, D), jnp.float32),
            pltpu.SemaphoreType.DMA((NBUF,)),
            pltpu.SemaphoreType.DMA((NBUF,)),
        ],
    )
    def body(idx_ref, seq_ref, tbl_ref, out_ref,
             idx_v, rows_v, seq_v, tbl_sh, gsem, psem):
        sid = lax.axis_index("s")
        wid = sid * NC + lax.axis_index("c")
        base = wid * B_PER_W

        @pl.when(sid == 0)
        def _():
            pltpu.sync_copy(tbl_ref, tbl_sh)

        pltpu.sync_copy(idx_ref.at[wid], idx_v)
        pltpu.sync_copy(seq_ref, seq_v)
        seq = seq_v[...]
        lane = lax.iota(jnp.int32, L)

        def xform(j, _):
            for k in range(VECS):
                pos0 = base + j * CHUNK + k * L
                t = lax.rem(pos0 + lane, NT)
                v = idx_v[j, pl.ds(k * L, L)]
                idx_v[j, pl.ds(k * L, L)] = jnp.where(t < seq, v + 1, 0)
            return 0

        lax.fori_loop(0, NCHUNK, xform, 0)
        plsc.subcore_barrier()

        def gather_cp(b, j):
            return pltpu.make_async_copy(
                tbl_sh.at[idx_v.at[j]], rows_v.at[b], gsem.at[b])

        def put_cp(b, j):
            return pltpu.make_async_copy(
                rows_v.at[b], out_ref.at[pl.ds(base + j * CHUNK, CHUNK)],
                psem.at[b])

        for b in range(NBUF):
            gather_cp(b, b).start()

        def round_body(r, _):
            for b in range(NBUF):
                j = r * NBUF + b
                gather_cp(b, j).wait()
                put_cp(b, j).start()
                jn = j + NBUF

                @pl.when(jn < NCHUNK)
                def _():
                    put_cp(b, j).wait()
                    gather_cp(b, jn).start()
            return 0

        lax.fori_loop(0, ROUNDS, round_body, 0)

        for b in range(NBUF):
            put_cp(b, (ROUNDS - 1) * NBUF + b).wait()

    return body(idx_hbm, seq_hbm, table_hbm)


def kernel(text, seq_len, text_embed, text_embed_ko):
    del text_embed
    idx = text.reshape(NW, NCHUNK, CHUNK).astype(jnp.int32)
    seq = jnp.full((L,), seq_len, dtype=jnp.int32)
    out = _sc_gather(idx, seq, text_embed_ko)
    return out.reshape(BATCH, NT, D)


# speedup vs baseline: 1.0185x; 1.0107x over previous
import functools

import jax
import jax.numpy as jnp
from jax import lax
from jax.experimental import pallas as pl
from jax.experimental.pallas import tpu as pltpu
from jax.experimental.pallas import tpu_sc as plsc

BATCH = 1024
NT = 200
D = 128
NC, NS, L = 2, 16, 16
NW = NC * NS
R_PER_W = BATCH // NW
IDX_PER_W = R_PER_W * NT
NVEC = IDX_PER_W // L
OVERLAP = (NT // L) * L - (NT - L)
G1 = 128
G2 = NT - G1
NBUF = 4
ROUNDS = R_PER_W // NBUF
TBL_ROWS = 158


def _sc_gather(text_hbm, seq_hbm, table_hbm):
    mesh = plsc.VectorSubcoreMesh(core_axis_name="c", subcore_axis_name="s")

    @functools.partial(
        pl.kernel,
        out_type=jax.ShapeDtypeStruct((BATCH * NT, D), jnp.float32),
        mesh=mesh,
        scratch_types=[
            pltpu.VMEM((R_PER_W, NT), jnp.int32),
            pltpu.VMEM((NBUF, NT, D), jnp.float32),
            pltpu.VMEM((L,), jnp.int32),
            pltpu.VMEM_SHARED((TBL_ROWS, D), jnp.float32),
            pltpu.SemaphoreType.DMA((NBUF,)),
            pltpu.SemaphoreType.DMA((NBUF,)),
        ],
    )
    def body(text_ref, seq_ref, tbl_ref, out_ref,
             idx_v, rows_v, seq_v, tbl_sh, gsem, psem):
        sid = lax.axis_index("s")
        wid = sid * NC + lax.axis_index("c")
        rbase = wid * R_PER_W

        @pl.when(sid == 0)
        def _():
            pltpu.sync_copy(tbl_ref, tbl_sh)

        pltpu.sync_copy(text_ref.at[pl.ds(rbase, R_PER_W)], idx_v)
        pltpu.sync_copy(seq_ref, seq_v)
        seq = seq_v[...]
        lane = lax.iota(jnp.int32, L)

        def xform(r, _):
            for j in range(NT // L):
                c = j * L + lane
                v = idx_v[r, pl.ds(j * L, L)]
                idx_v[r, pl.ds(j * L, L)] = jnp.where(c < seq, v + 1, 0)
            c = (NT - L) + lane
            v = idx_v[r, pl.ds(NT - L, L)]
            t = jnp.where(c < seq, v + 1, 0)
            idx_v[r, pl.ds(NT - L, L)] = jnp.where(lane >= OVERLAP, t, v)
            return 0

        lax.fori_loop(0, R_PER_W, xform, 0)
        plsc.subcore_barrier()

        def g1_cp(b, r):
            return pltpu.make_async_copy(
                tbl_sh.at[idx_v.at[r, pl.ds(0, G1)]],
                rows_v.at[b, pl.ds(0, G1)], gsem.at[b])

        def g2_cp(b, r):
            return pltpu.make_async_copy(
                tbl_sh.at[idx_v.at[r, pl.ds(G1, G2)]],
                rows_v.at[b, pl.ds(G1, G2)], gsem.at[b])

        def start_gathers(b, r):
            g1_cp(b, r).start()
            g2_cp(b, r).start()

        def wait_gathers(b, r):
            g1_cp(b, r).wait()
            g2_cp(b, r).wait()

        def put_cp(b, r):
            return pltpu.make_async_copy(
                rows_v.at[b], out_ref.at[pl.ds((rbase + r) * NT, NT)],
                psem.at[b])

        for b in range(NBUF):
            start_gathers(b, b)

        def round_body(q, _):
            for b in range(NBUF):
                r = q * NBUF + b
                wait_gathers(b, r)
                put_cp(b, r).start()
                rn = r + NBUF

                @pl.when(rn < R_PER_W)
                def _():
                    put_cp(b, r).wait()
                    start_gathers(b, rn)
            return 0

        lax.fori_loop(0, ROUNDS, round_body, 0)

        for b in range(NBUF):
            put_cp(b, (ROUNDS - 1) * NBUF + b).wait()

    return body(text_hbm, seq_hbm, table_hbm)


def kernel(text, seq_len, text_embed, text_embed_ko):
    del text_embed
    seq = jnp.full((L,), seq_len, dtype=jnp.int32)
    out = _sc_gather(text.astype(jnp.int32), seq, text_embed_ko)
    return out.reshape(BATCH, NT, D)
